# Initial kernel scaffold; baseline (speedup 1.0000x reference)
import jax
import jax.numpy as jnp
from jax.experimental import pallas as pl


def kernel(conf_data, loc_data, priors, targets):
    raise NotImplementedError("write your pallas kernel here")


# speedup vs baseline: 22.7899x; 22.7899x over previous
import functools

import jax
import jax.numpy as jnp
from jax import lax
from jax.experimental import pallas as pl

_NUM_CLASSES = 9
_NEGPOS_RATIO = 3
_VAR0 = 0.1
_VAR1 = 0.2
_THRESH = 0.5

_R = 8
_L = 1152
_PPAD = _R * _L


def _smooth_l1(x):
    ax = jnp.abs(x)
    return jnp.where(ax < 1.0, 0.5 * ax * ax, ax - 0.5)


def _body(num_priors, num_truths, tgt_ref, pri_ref, conf_ref, loc_ref, out_ref):
    f32 = jnp.float32
    gidx = (lax.broadcasted_iota(jnp.int32, (_R, _L), 0) * _L
            + lax.broadcasted_iota(jnp.int32, (_R, _L), 1))

    pcx = pri_ref[0]
    pcy = pri_ref[1]
    pw = pri_ref[2]
    ph = pri_ref[3]
    px1 = pcx - pw / 2.0
    py1 = pcy - ph / 2.0
    px2 = pcx + pw / 2.0
    py2 = pcy + ph / 2.0
    area_b = (px2 - px1) * (py2 - py1)

    tgt = [[tgt_ref[0, 0, o, j] for j in range(5)] for o in range(num_truths)]

    bto = None
    lab = None
    m1 = m2 = m3 = m4 = None
    best_prior = []
    for o in range(num_truths):
        t1, t2, t3, t4, t5 = tgt[o]
        ltx = jnp.maximum(t1, px1)
        lty = jnp.maximum(t2, py1)
        rbx = jnp.minimum(t3, px2)
        rby = jnp.minimum(t4, py2)
        wx = jnp.clip(rbx - ltx, 0.0, None)
        wy = jnp.clip(rby - lty, 0.0, None)
        inter = wx * wy
        area_a = (t3 - t1) * (t4 - t2)
        iou = inter / (area_a + area_b - inter)
        mo = jnp.max(iou)
        bp = jnp.min(jnp.where(iou == mo, gidx, jnp.int32(2 ** 30)))
        best_prior.append(bp)
        if o == 0:
            bto = iou
            lab = jnp.full((_R, _L), t5, f32)
            m1 = jnp.full((_R, _L), t1, f32)
            m2 = jnp.full((_R, _L), t2, f32)
            m3 = jnp.full((_R, _L), t3, f32)
            m4 = jnp.full((_R, _L), t4, f32)
        else:
            upd = iou > bto
            bto = jnp.where(upd, iou, bto)
            lab = jnp.where(upd, t5, lab)
            m1 = jnp.where(upd, t1, m1)
            m2 = jnp.where(upd, t2, m2)
            m3 = jnp.where(upd, t3, m3)
            m4 = jnp.where(upd, t4, m4)

    for o in range(num_truths):
        t1, t2, t3, t4, t5 = tgt[o]
        hit = gidx == best_prior[o]
        bto = jnp.where(hit, 2.0, bto)
        lab = jnp.where(hit, t5, lab)
        m1 = jnp.where(hit, t1, m1)
        m2 = jnp.where(hit, t2, m2)
        m3 = jnp.where(hit, t3, m3)
        m4 = jnp.where(hit, t4, m4)

    conf = jnp.where(bto < _THRESH, 0, lab.astype(jnp.int32) + 1)
    pos = conf > 0
    soft = jnp.minimum(bto, 1.0)

    g1 = ((m1 + m3) / 2.0 - pcx) / (_VAR0 * pw)
    g2 = ((m2 + m4) / 2.0 - pcy) / (_VAR0 * ph)
    g3 = jnp.log((m3 - m1) / pw) / _VAR1
    g4 = jnp.log((m4 - m2) / ph) / _VAR1
    zero = jnp.zeros((_R, _L), f32)
    loss_l = (jnp.sum(jnp.where(pos, _smooth_l1(loc_ref[0, 0] - g1), zero))
              + jnp.sum(jnp.where(pos, _smooth_l1(loc_ref[0, 1] - g2), zero))
              + jnp.sum(jnp.where(pos, _smooth_l1(loc_ref[0, 2] - g3), zero))
              + jnp.sum(jnp.where(pos, _smooth_l1(loc_ref[0, 3] - g4), zero)))

    rows = [conf_ref[0, c] for c in range(_NUM_CLASSES)]
    m9 = rows[0]
    for c in range(1, _NUM_CLASSES):
        m9 = jnp.maximum(m9, rows[c])
    s = jnp.exp(rows[0] - m9)
    for c in range(1, _NUM_CLASSES):
        s = s + jnp.exp(rows[c] - m9)
    lse = m9 + jnp.log(s)

    valid = gidx < num_priors
    lc = jnp.where(jnp.logical_and(valid, jnp.logical_not(pos)),
                   lse - rows[0], zero)

    logit_sel = rows[0]
    for c in range(1, _NUM_CLASSES):
        logit_sel = jnp.where(conf == c, rows[c], logit_sel)
    alpha = jnp.where(conf <= 1, 0.5, 1.0)
    pos_term = jnp.sum(jnp.where(pos, soft * alpha * (lse - logit_sel), zero))

    num_pos = jnp.sum(pos.astype(jnp.int32))
    k = jnp.minimum(_NEGPOS_RATIO * num_pos, num_priors - 1)

    li = lax.bitcast_convert_type(lc, jnp.int32)

    def radix_step(t, cand):
        bit = jnp.int32(30) - t
        cand2 = jnp.bitwise_or(cand, jnp.left_shift(jnp.int32(1), bit))
        cnt = jnp.sum((li >= cand2).astype(jnp.int32))
        return jnp.where(cnt >= k, cand2, cand)

    vk_bits = lax.fori_loop(0, 31, radix_step, jnp.int32(0))
    vk = lax.bitcast_convert_type(vk_bits, f32)
    gt = lc > vk
    cnt_gt = jnp.sum(gt.astype(jnp.int32))
    sum_gt = jnp.sum(jnp.where(gt, lc, zero))
    neg_sum = sum_gt + (k - cnt_gt).astype(f32) * vk

    lane = lax.broadcasted_iota(jnp.int32, (1, 128), 1)
    vec = jnp.where(lane == 0, pos_term,
                    jnp.where(lane == 1, neg_sum,
                              jnp.where(lane == 2, loss_l,
                                        jnp.where(lane == 3,
                                                  num_pos.astype(f32), 0.0))))
    out_ref[0] = vec


def kernel(conf_data, loc_data, priors, targets):
    B, P, C = conf_data.shape
    O = targets.shape[1]
    pad = _PPAD - P
    conf_r = jnp.pad(jnp.transpose(conf_data, (0, 2, 1)),
                     ((0, 0), (0, 0), (0, pad))).reshape(B, C, _R, _L)
    loc_r = jnp.pad(jnp.transpose(loc_data, (0, 2, 1)),
                    ((0, 0), (0, 0), (0, pad))).reshape(B, 4, _R, _L)
    pri_r = jnp.pad(jnp.transpose(priors, (1, 0)),
                    ((0, 0), (0, pad))).reshape(4, _R, _L)
    tgt_r = targets.reshape(B, 1, O, 5)

    parts = pl.pallas_call(
        functools.partial(_body, P, O),
        grid=(B,),
        in_specs=[
            pl.BlockSpec((1, 1, O, 5), lambda i: (i, 0, 0, 0)),
            pl.BlockSpec((4, _R, _L), lambda i: (0, 0, 0)),
            pl.BlockSpec((1, C, _R, _L), lambda i: (i, 0, 0, 0)),
            pl.BlockSpec((1, 4, _R, _L), lambda i: (i, 0, 0, 0)),
        ],
        out_specs=pl.BlockSpec((1, 1, 128), lambda i: (i, 0, 0)),
        out_shape=jax.ShapeDtypeStruct((B, 1, 128), jnp.float32),
    )(tgt_r, pri_r, conf_r, loc_r)

    sums = jnp.sum(parts[:, 0, :4], axis=0)
    loss_c_total = sums[0] + 0.5 * sums[1]
    loss_l_total = sums[2]
    n_pos_total = sums[3]
    N = jnp.where(n_pos_total > 0, n_pos_total, jnp.float32(B))
    return (loss_c_total / N, loss_l_total / N)


# speedup vs baseline: 54.2948x; 2.3824x over previous
import functools

import jax
import jax.numpy as jnp
from jax import lax
from jax.experimental import pallas as pl

_NUM_CLASSES = 9
_NEGPOS_RATIO = 3
_VAR0 = 0.1
_VAR1 = 0.2
_THRESH = 0.5

_R = 8
_L = 1152
_PPAD = _R * _L


def _smooth_l1(x):
    ax = jnp.abs(x)
    return jnp.where(ax < 1.0, 0.5 * ax * ax, ax - 0.5)


def _match_body(num_priors, num_truths, tgt_ref, pri_ref, conf_ref, loc_ref,
                lc_ref, part_ref):
    f32 = jnp.float32
    gidx = (lax.broadcasted_iota(jnp.int32, (_R, _L), 0) * _L
            + lax.broadcasted_iota(jnp.int32, (_R, _L), 1))

    pcx = pri_ref[0]
    pcy = pri_ref[1]
    pw = pri_ref[2]
    ph = pri_ref[3]
    px1 = pcx - pw / 2.0
    py1 = pcy - ph / 2.0
    px2 = pcx + pw / 2.0
    py2 = pcy + ph / 2.0
    area_b = (px2 - px1) * (py2 - py1)

    tgt = [[tgt_ref[0, 0, o, j] for j in range(5)] for o in range(num_truths)]
    O = num_truths

    t1c = tgt_ref[0, 0][:, 0].reshape(O, 1, 1)
    t2c = tgt_ref[0, 0][:, 1].reshape(O, 1, 1)
    t3c = tgt_ref[0, 0][:, 2].reshape(O, 1, 1)
    t4c = tgt_ref[0, 0][:, 3].reshape(O, 1, 1)
    ltx = jnp.maximum(t1c, px1[None])
    lty = jnp.maximum(t2c, py1[None])
    rbx = jnp.minimum(t3c, px2[None])
    rby = jnp.minimum(t4c, py2[None])
    inter = (jnp.clip(rbx - ltx, 0.0, None) * jnp.clip(rby - lty, 0.0, None))
    area_a = (t3c - t1c) * (t4c - t2c)
    iou3 = inter / (area_a + area_b[None] - inter)

    mo = jnp.max(jnp.max(iou3, axis=1), axis=1)
    eqm = iou3 == mo[:, None, None]
    big = jnp.int32(2 ** 30)
    bp = jnp.min(jnp.min(jnp.where(eqm, gidx[None], big), axis=1), axis=1)
    tidx3 = lax.broadcasted_iota(jnp.int32, (O, _R, _L), 0)
    hit = gidx[None] == bp[:, None, None]
    forced_o = jnp.max(jnp.where(hit, tidx3, -1), axis=0)

    bto = iou3[0]
    lab = jnp.full((_R, _L), tgt[0][4], f32)
    m1 = jnp.full((_R, _L), tgt[0][0], f32)
    m2 = jnp.full((_R, _L), tgt[0][1], f32)
    m3 = jnp.full((_R, _L), tgt[0][2], f32)
    m4 = jnp.full((_R, _L), tgt[0][3], f32)
    for o in range(1, O):
        t1, t2, t3, t4, t5 = tgt[o]
        iou = iou3[o]
        upd = iou > bto
        bto = jnp.where(upd, iou, bto)
        lab = jnp.where(upd, t5, lab)
        m1 = jnp.where(upd, t1, m1)
        m2 = jnp.where(upd, t2, m2)
        m3 = jnp.where(upd, t3, m3)
        m4 = jnp.where(upd, t4, m4)
    forced = forced_o >= 0
    bto = jnp.where(forced, 2.0, bto)
    for o in range(O):
        t1, t2, t3, t4, t5 = tgt[o]
        sel = forced_o == o
        lab = jnp.where(sel, t5, lab)
        m1 = jnp.where(sel, t1, m1)
        m2 = jnp.where(sel, t2, m2)
        m3 = jnp.where(sel, t3, m3)
        m4 = jnp.where(sel, t4, m4)

    conf = jnp.where(bto < _THRESH, 0, lab.astype(jnp.int32) + 1)
    pos = conf > 0
    soft = jnp.minimum(bto, 1.0)

    g1 = ((m1 + m3) / 2.0 - pcx) / (_VAR0 * pw)
    g2 = ((m2 + m4) / 2.0 - pcy) / (_VAR0 * ph)
    g3 = jnp.log((m3 - m1) / pw) / _VAR1
    g4 = jnp.log((m4 - m2) / ph) / _VAR1
    zero = jnp.zeros((_R, _L), f32)
    loss_l = (jnp.sum(jnp.where(pos, _smooth_l1(loc_ref[0, 0] - g1), zero))
              + jnp.sum(jnp.where(pos, _smooth_l1(loc_ref[0, 1] - g2), zero))
              + jnp.sum(jnp.where(pos, _smooth_l1(loc_ref[0, 2] - g3), zero))
              + jnp.sum(jnp.where(pos, _smooth_l1(loc_ref[0, 3] - g4), zero)))

    rows = [conf_ref[0, c] for c in range(_NUM_CLASSES)]
    m9 = rows[0]
    for c in range(1, _NUM_CLASSES):
        m9 = jnp.maximum(m9, rows[c])
    s = jnp.exp(rows[0] - m9)
    for c in range(1, _NUM_CLASSES):
        s = s + jnp.exp(rows[c] - m9)
    lse = m9 + jnp.log(s)

    valid = gidx < num_priors
    lc_ref[0] = jnp.where(jnp.logical_and(valid, jnp.logical_not(pos)),
                          lse - rows[0], zero)

    logit_sel = rows[0]
    for c in range(1, _NUM_CLASSES):
        logit_sel = jnp.where(conf == c, rows[c], logit_sel)
    alpha = jnp.where(conf <= 1, 0.5, 1.0)
    pos_term = jnp.sum(jnp.where(pos, soft * alpha * (lse - logit_sel), zero))
    num_pos = jnp.sum(pos.astype(jnp.int32))

    lane = lax.broadcasted_iota(jnp.int32, (1, 128), 1)
    vec = jnp.where(lane == 0, pos_term,
                    jnp.where(lane == 1, loss_l,
                              jnp.where(lane == 2, num_pos.astype(f32), 0.0)))
    part_ref[0] = vec


def _mining_body(num_priors, num_images, lc_ref, part_ref, out_ref):
    f32 = jnp.float32
    B = num_images
    lc = lc_ref[...]
    li = lax.bitcast_convert_type(lc, jnp.int32)
    np_vec = part_ref[:, 0, 2]
    k = jnp.minimum(_NEGPOS_RATIO * np_vec.astype(jnp.int32),
                    num_priors - 1)

    def radix_step(t, cand):
        bit = jnp.int32(30) - t
        cand2 = jnp.bitwise_or(cand, jnp.left_shift(jnp.int32(1), bit))
        ge = (li >= cand2[:, None, None]).astype(jnp.int32)
        cnt = jnp.sum(jnp.sum(ge, axis=1), axis=1)
        return jnp.where(cnt >= k, cand2, cand)

    vk_bits = lax.fori_loop(0, 31, radix_step, jnp.zeros((B,), jnp.int32))
    vk = lax.bitcast_convert_type(vk_bits, f32)
    gt = lc > vk[:, None, None]
    cnt_gt = jnp.sum(jnp.sum(gt.astype(jnp.int32), axis=1), axis=1)
    sum_gt = jnp.sum(jnp.sum(jnp.where(gt, lc, 0.0), axis=1), axis=1)
    neg_sum = sum_gt + (k - cnt_gt).astype(f32) * vk

    loss_c_total = jnp.sum(part_ref[:, 0, 0]) + 0.5 * jnp.sum(neg_sum)
    loss_l_total = jnp.sum(part_ref[:, 0, 1])
    n_pos_total = jnp.sum(np_vec)
    N = jnp.where(n_pos_total > 0, n_pos_total, jnp.float32(B))

    lane = lax.broadcasted_iota(jnp.int32, (1, 128), 1)
    out_ref[...] = jnp.where(lane == 0, loss_c_total / N,
                             jnp.where(lane == 1, loss_l_total / N, 0.0))


def kernel(conf_data, loc_data, priors, targets):
    B, P, C = conf_data.shape
    O = targets.shape[1]
    pad = _PPAD - P
    conf_r = jnp.pad(jnp.transpose(conf_data, (0, 2, 1)),
                     ((0, 0), (0, 0), (0, pad))).reshape(B, C, _R, _L)
    loc_r = jnp.pad(jnp.transpose(loc_data, (0, 2, 1)),
                    ((0, 0), (0, 0), (0, pad))).reshape(B, 4, _R, _L)
    pri_r = jnp.pad(jnp.transpose(priors, (1, 0)),
                    ((0, 0), (0, pad))).reshape(4, _R, _L)
    tgt_r = targets.reshape(B, 1, O, 5)

    lc_all, parts = pl.pallas_call(
        functools.partial(_match_body, P, O),
        grid=(B,),
        in_specs=[
            pl.BlockSpec((1, 1, O, 5), lambda i: (i, 0, 0, 0)),
            pl.BlockSpec((4, _R, _L), lambda i: (0, 0, 0)),
            pl.BlockSpec((1, C, _R, _L), lambda i: (i, 0, 0, 0)),
            pl.BlockSpec((1, 4, _R, _L), lambda i: (i, 0, 0, 0)),
        ],
        out_specs=[
            pl.BlockSpec((1, _R, _L), lambda i: (i, 0, 0)),
            pl.BlockSpec((1, 1, 128), lambda i: (i, 0, 0)),
        ],
        out_shape=[
            jax.ShapeDtypeStruct((B, _R, _L), jnp.float32),
            jax.ShapeDtypeStruct((B, 1, 128), jnp.float32),
        ],
    )(tgt_r, pri_r, conf_r, loc_r)

    out = pl.pallas_call(
        functools.partial(_mining_body, P, B),
        out_shape=jax.ShapeDtypeStruct((1, 128), jnp.float32),
    )(lc_all, parts)

    return (out[0, 0], out[0, 1])
